# baseline (speedup 1.0000x reference)
import jax
import jax.numpy as jnp
from jax.experimental import pallas as pl
from jax.experimental.pallas import tpu as pltpu

B, N, D, OUT = 4, 100000, 128, 10
TILE_N = 20000
NT = N // TILE_N
BNT = B * NT
NCH = 10
CH = TILE_N // NCH


def _gcn_pool_kernel(h_ref, wab_ref, wcd_ref, pooled_ref, c1_ref, c3_ref):
    s = pl.program_id(0)

    @pl.when(s == 0)
    def _init():
        pooled_ref[:, :] = jnp.zeros((8, D), jnp.float32)
        c1_ref[:, :] = jnp.zeros((TILE_N, D), jnp.bfloat16)
        c3_ref[:, :] = jnp.zeros((TILE_N, D), jnp.bfloat16)

    def stage(in_left, in_right, w_ref):
        inp = jnp.concatenate([in_left, in_right], axis=1)
        return jnp.dot(inp, w_ref[:, :], preferred_element_type=jnp.float32)

    partial = jnp.zeros((1, D), jnp.float32)
    h2s = []
    for i in range(NCH):
        r = slice(i * CH, (i + 1) * CH)
        x_i = h_ref[0, r, :].astype(jnp.bfloat16)
        o_a = jnp.maximum(stage(x_i, c1_ref[r, :], wab_ref)
                          .astype(jnp.bfloat16), 0.0)
        c1_ref[r, :] = o_a[:, :D]
        h2s.append(o_a[:, D:])
    for i in range(NCH):
        r = slice(i * CH, (i + 1) * CH)
        o_b = stage(h2s[i], c3_ref[r, :], wcd_ref)
        c3_ref[r, :] = jnp.maximum(o_b[:, :D].astype(jnp.bfloat16), 0.0)
        h4_i = jnp.maximum(o_b[:, D:], 0.0)
        partial = partial + jnp.sum(h4_i, axis=0, keepdims=True)

    @pl.when(s >= 2)
    def _pool():
        b4 = (s - 2) // NT
        rows = jax.lax.broadcasted_iota(jnp.int32, (8, D), 0)
        pooled_ref[:, :] = jnp.where(rows == b4, pooled_ref[:, :] + partial,
                                     pooled_ref[:, :])


def _mlp_kernel(pooled_ref, c1w_ref, c1b_ref, c2w_ref, c2b_ref,
                c3w_ref, c3b_ref, out_ref):
    acc = pooled_ref[0:B, :]
    y = jnp.maximum(jnp.dot(acc, c1w_ref[:, :],
                            preferred_element_type=jnp.float32)
                    + c1b_ref[:, :], 0.0)
    y = jnp.maximum(jnp.dot(y, c2w_ref[:, :],
                            preferred_element_type=jnp.float32)
                    + c2b_ref[:, :], 0.0)
    out_ref[:, :] = (jnp.dot(y, c3w_ref[:, :],
                             preferred_element_type=jnp.float32)
                     + c3b_ref[:, :])


def _blkdiag(w_top, w_bot):
    z = jnp.zeros((D, D), jnp.bfloat16)
    return jnp.concatenate(
        [jnp.concatenate([w_top.astype(jnp.bfloat16), z], axis=1),
         jnp.concatenate([z, w_bot.astype(jnp.bfloat16)], axis=1)], axis=0)


def kernel(h_0, W_in, W_h1, W_h2, W_out, C1_w, C1_b, C2_w, C2_b, C3_w, C3_b):
    w_ab = _blkdiag(W_in, W_h1)
    w_cd = _blkdiag(W_h2, W_out)

    def x_map(s):
        t = jnp.minimum(s, BNT - 1)
        return (t // NT, t % NT, 0)

    pooled = pl.pallas_call(
        _gcn_pool_kernel,
        grid=(BNT + 2,),
        in_specs=[
            pl.BlockSpec((1, TILE_N, D), x_map),
            pl.BlockSpec((2 * D, 2 * D), lambda s: (0, 0)),
            pl.BlockSpec((2 * D, 2 * D), lambda s: (0, 0)),
        ],
        out_specs=pl.BlockSpec((8, D), lambda s: (0, 0)),
        out_shape=jax.ShapeDtypeStruct((8, D), jnp.float32),
        scratch_shapes=[pltpu.VMEM((TILE_N, D), jnp.bfloat16),
                        pltpu.VMEM((TILE_N, D), jnp.bfloat16)],
        compiler_params=pltpu.CompilerParams(
            dimension_semantics=("arbitrary",)),
    )(h_0, w_ab, w_cd)

    return pl.pallas_call(
        _mlp_kernel,
        in_specs=[
            pl.BlockSpec((8, D), lambda: (0, 0)),
            pl.BlockSpec((D, D), lambda: (0, 0)),
            pl.BlockSpec((1, D), lambda: (0, 0)),
            pl.BlockSpec((D, D), lambda: (0, 0)),
            pl.BlockSpec((1, D), lambda: (0, 0)),
            pl.BlockSpec((D, OUT), lambda: (0, 0)),
            pl.BlockSpec((1, OUT), lambda: (0, 0)),
        ],
        out_specs=pl.BlockSpec((B, OUT), lambda: (0, 0)),
        out_shape=jax.ShapeDtypeStruct((B, OUT), jnp.float32),
    )(pooled, C1_w, C1_b.reshape(1, D), C2_w, C2_b.reshape(1, D),
      C3_w, C3_b.reshape(1, OUT))


# speedup vs baseline: 1.0200x; 1.0200x over previous
import jax
import jax.numpy as jnp
from jax.experimental import pallas as pl
from jax.experimental.pallas import tpu as pltpu

B, N, D, OUT = 4, 100000, 128, 10
TILE_N = 10000
NT = N // TILE_N
BNT = B * NT
NCH = 5
CH = TILE_N // NCH


def _gcn_pool_kernel(h_ref, wab_ref, wcd_ref, pooled_ref, c1_ref, c3_ref):
    s = pl.program_id(0)

    @pl.when(s == 0)
    def _init():
        pooled_ref[:, :] = jnp.zeros((8, D), jnp.float32)
        c1_ref[:, :] = jnp.zeros((TILE_N, D), jnp.bfloat16)
        c3_ref[:, :] = jnp.zeros((TILE_N, D), jnp.bfloat16)

    def stage(in_left, in_right, w_ref):
        inp = jnp.concatenate([in_left, in_right], axis=1)
        return jnp.dot(inp, w_ref[:, :], preferred_element_type=jnp.float32)

    partial = jnp.zeros((1, D), jnp.float32)
    h2s = [None] * NCH

    def do_a(i):
        r = slice(i * CH, (i + 1) * CH)
        x_i = h_ref[0, r, :].astype(jnp.bfloat16)
        o_a = jnp.maximum(stage(x_i, c1_ref[r, :], wab_ref)
                          .astype(jnp.bfloat16), 0.0)
        c1_ref[r, :] = o_a[:, :D]
        h2s[i] = o_a[:, D:]

    def do_b(i):
        nonlocal partial
        r = slice(i * CH, (i + 1) * CH)
        o_b = stage(h2s[i], c3_ref[r, :], wcd_ref)
        c3_ref[r, :] = jnp.maximum(o_b[:, :D].astype(jnp.bfloat16), 0.0)
        h4_i = jnp.maximum(o_b[:, D:], 0.0)
        partial = partial + jnp.sum(h4_i, axis=0, keepdims=True)

    do_a(0)
    for i in range(1, NCH):
        do_a(i)
        do_b(i - 1)
    do_b(NCH - 1)

    @pl.when(s >= 2)
    def _pool():
        b4 = (s - 2) // NT
        rows = jax.lax.broadcasted_iota(jnp.int32, (8, D), 0)
        pooled_ref[:, :] = jnp.where(rows == b4, pooled_ref[:, :] + partial,
                                     pooled_ref[:, :])

    @pl.when(s == BNT - 1)
    def _tail():
        p2 = jnp.zeros((1, D), jnp.float32)
        for i in range(NCH):
            r = slice(i * CH, (i + 1) * CH)
            c1r = c1_ref[r, :]
            h2r = jnp.maximum(stage(c1r, c1r, wab_ref)[:, D:]
                              .astype(jnp.bfloat16), 0.0)
            ob1 = stage(h2r, c3_ref[r, :], wcd_ref)
            h3r = jnp.maximum(ob1[:, :D].astype(jnp.bfloat16), 0.0)
            h4a = jnp.maximum(ob1[:, D:], 0.0)
            h4b = jnp.maximum(stage(h3r, h3r, wcd_ref)[:, D:], 0.0)
            p2 = (p2 + jnp.sum(h4a, axis=0, keepdims=True)
                  + jnp.sum(h4b, axis=0, keepdims=True))
        rows = jax.lax.broadcasted_iota(jnp.int32, (8, D), 0)
        pooled_ref[:, :] = jnp.where(rows == B - 1, pooled_ref[:, :] + p2,
                                     pooled_ref[:, :])


def _mlp_kernel(pooled_ref, c1w_ref, c1b_ref, c2w_ref, c2b_ref,
                c3w_ref, c3b_ref, out_ref):
    acc = pooled_ref[0:B, :]
    y = jnp.maximum(jnp.dot(acc, c1w_ref[:, :],
                            preferred_element_type=jnp.float32)
                    + c1b_ref[:, :], 0.0)
    y = jnp.maximum(jnp.dot(y, c2w_ref[:, :],
                            preferred_element_type=jnp.float32)
                    + c2b_ref[:, :], 0.0)
    out_ref[:, :] = (jnp.dot(y, c3w_ref[:, :],
                             preferred_element_type=jnp.float32)
                     + c3b_ref[:, :])


def _blkdiag(w_top, w_bot):
    z = jnp.zeros((D, D), jnp.bfloat16)
    return jnp.concatenate(
        [jnp.concatenate([w_top.astype(jnp.bfloat16), z], axis=1),
         jnp.concatenate([z, w_bot.astype(jnp.bfloat16)], axis=1)], axis=0)


def kernel(h_0, W_in, W_h1, W_h2, W_out, C1_w, C1_b, C2_w, C2_b, C3_w, C3_b):
    w_ab = _blkdiag(W_in, W_h1)
    w_cd = _blkdiag(W_h2, W_out)

    def x_map(s):
        return (s // NT, s % NT, 0)

    pooled = pl.pallas_call(
        _gcn_pool_kernel,
        grid=(BNT,),
        in_specs=[
            pl.BlockSpec((1, TILE_N, D), x_map),
            pl.BlockSpec((2 * D, 2 * D), lambda s: (0, 0)),
            pl.BlockSpec((2 * D, 2 * D), lambda s: (0, 0)),
        ],
        out_specs=pl.BlockSpec((8, D), lambda s: (0, 0)),
        out_shape=jax.ShapeDtypeStruct((8, D), jnp.float32),
        scratch_shapes=[pltpu.VMEM((TILE_N, D), jnp.bfloat16),
                        pltpu.VMEM((TILE_N, D), jnp.bfloat16)],
        compiler_params=pltpu.CompilerParams(
            dimension_semantics=("arbitrary",)),
    )(h_0, w_ab, w_cd)

    return pl.pallas_call(
        _mlp_kernel,
        in_specs=[
            pl.BlockSpec((8, D), lambda: (0, 0)),
            pl.BlockSpec((D, D), lambda: (0, 0)),
            pl.BlockSpec((1, D), lambda: (0, 0)),
            pl.BlockSpec((D, D), lambda: (0, 0)),
            pl.BlockSpec((1, D), lambda: (0, 0)),
            pl.BlockSpec((D, OUT), lambda: (0, 0)),
            pl.BlockSpec((1, OUT), lambda: (0, 0)),
        ],
        out_specs=pl.BlockSpec((B, OUT), lambda: (0, 0)),
        out_shape=jax.ShapeDtypeStruct((B, OUT), jnp.float32),
    )(pooled, C1_w, C1_b.reshape(1, D), C2_w, C2_b.reshape(1, D),
      C3_w, C3_b.reshape(1, OUT))
